# baseline (speedup 1.0000x reference)
import functools

import jax
import jax.numpy as jnp
from jax import lax
from jax.experimental import pallas as pl
from jax.experimental.pallas import tpu as pltpu
from jax.experimental.pallas import tpu_sc as plsc

_K = 8192
_D = 256
_N = 16384
_TR = 256
_NT = _N // _TR


_WLO = (0, 2736, 5472)
_WHI = (2736, 5472, _K)


def _argmin_body(flat_ref, embb_ref, fn_ref, en_ref, idx_ref, loss_ref):
    import functools as _ft

    i = pl.program_id(0)

    @pl.when(i == 0)
    def _():
        loss_ref[0, 0] = 0.0

    flat = flat_ref[...]
    fn = fn_ref[...][:, None]
    mm = lax.dot_general(
        flat.astype(jnp.bfloat16), embb_ref[...],
        (((1,), (1,)), ((), ())),
        preferred_element_type=jnp.float32)
    dist = (fn + en_ref[...][None, :]) - 2.0 * mm

    inf = jnp.float32(jnp.inf)
    lane = lax.broadcasted_iota(jnp.int32, (_TR, 128), 1)
    inf_acc = jnp.full((_TR, 128), inf, jnp.float32)

    def tile_window(t):
        if t == 21:
            return [(0, lane < 48), (1, lane >= 48)]
        if t == 42:
            return [(1, lane < 96), (2, lane >= 96)]
        return [(0 if t < 21 else (1 if t < 42 else 2), None)]

    acc = [inf_acc, inf_acc, inf_acc]
    for t in range(64):
        blk = dist[:, t * 128:(t + 1) * 128]
        for w, msk in tile_window(t):
            piece = blk if msk is None else jnp.where(msk, blk, inf)
            acc[w] = jnp.minimum(acc[w], piece)
    wm = [jnp.min(a, axis=1) for a in acc]

    sel_v, sel_w = wm[0], jnp.zeros_like(wm[0], jnp.int32)
    acc_v = wm[0].astype(jnp.bfloat16).astype(jnp.float32)
    for w in (1, 2):
        pred = wm[w] < acc_v
        sel_v = jnp.where(pred, wm[w], sel_v)
        sel_w = jnp.where(pred, w, sel_w)
        acc_v = jnp.where(pred, wm[w], acc_v).astype(jnp.bfloat16).astype(jnp.float32)

    big = jnp.full((_TR, 128), _K, jnp.int32)
    iacc = [big, big, big]
    sv = sel_v[:, None]
    for t in range(64):
        blk = dist[:, t * 128:(t + 1) * 128]
        cand = jnp.where(blk == sv, lane + (t * 128), _K)
        for w, msk in tile_window(t):
            piece = cand if msk is None else jnp.where(msk, cand, _K)
            iacc[w] = jnp.minimum(iacc[w], piece)
    wi = [jnp.min(a, axis=1) for a in iacc]
    idx = jnp.where(sel_w == 0, wi[0], jnp.where(sel_w == 1, wi[1], wi[2]))
    idx_ref[...] = idx
    loss_ref[0, 0] += jnp.sum(sel_v)


_argmin_call = pl.pallas_call(
    _argmin_body,
    grid=(_NT,),
    in_specs=[
        pl.BlockSpec((_TR, _D), lambda i: (i, 0)),
        pl.BlockSpec((_K, _D), lambda i: (0, 0)),
        pl.BlockSpec((_TR,), lambda i: (i,)),
        pl.BlockSpec((_K,), lambda i: (0,)),
    ],
    out_specs=[
        pl.BlockSpec((_TR,), lambda i: (i,)),
        pl.BlockSpec(memory_space=pltpu.SMEM),
    ],
    out_shape=[
        jax.ShapeDtypeStruct((_N,), jnp.int32),
        jax.ShapeDtypeStruct((1, 1), jnp.float32),
    ],
)


_NC, _NS = 2, 16
_NW = _NC * _NS
_BPW = _N // _NW
_CH = 128
_NCH = _BPW // _CH


@functools.cache
def _sc_gather_call():
    @functools.partial(
        pl.kernel,
        mesh=plsc.VectorSubcoreMesh(core_axis_name="c", subcore_axis_name="s"),
        out_type=jax.ShapeDtypeStruct((_N, _D), jnp.float32),
        scratch_types=[
            pltpu.VMEM((_NCH, _CH), jnp.int32),
            pltpu.VMEM((_CH, _D), jnp.float32),
            pltpu.SemaphoreType.DMA,
        ],
    )
    def _sc_gather(table_hbm, idx_hbm, out_hbm, idx_v, rows_v, sem):
        wid = lax.axis_index("s") * _NC + lax.axis_index("c")
        base = wid * _BPW
        for c in range(_NCH):
            pltpu.sync_copy(idx_hbm.at[pl.ds(base + c * _CH, _CH)], idx_v.at[c])
            pltpu.async_copy(table_hbm.at[idx_v.at[c]], rows_v, sem).wait()
            pltpu.sync_copy(rows_v, out_hbm.at[pl.ds(base + c * _CH, _CH)])

    return _sc_gather


def kernel(z_e, embed_weight):
    flat = z_e.reshape(_N, _D)
    fn = jnp.sum(flat ** 2, axis=1)
    en = jnp.sum(embed_weight ** 2, axis=1)
    idx, loss_sum = _argmin_call(flat, embed_weight.astype(jnp.bfloat16), fn, en)
    quantized = _sc_gather_call()(embed_weight, idx)
    m = loss_sum[0, 0] / (_N * _D)
    loss = m + 0.25 * m
    return quantized.reshape(z_e.shape), loss, idx
